# baseline (speedup 1.0000x reference)
import jax
import jax.numpy as jnp
from jax.experimental import pallas as pl
from jax.experimental.pallas import tpu as pltpu

_BM = 512
_BK = 512


def _body(x_ref, s_ref, war_ref, wma_ref, ar_ref, ma_ref, b_ref, o_ref,
          xw_ref, sw_ref):
    i = pl.program_id(0)
    k = pl.program_id(1)
    nk = pl.num_programs(1)

    @pl.when(i == 0)
    def _premix():
        xw_ref[pl.ds(k * _BK, _BK), :] = jnp.dot(
            x_ref[...].astype(jnp.bfloat16), war_ref[...],
            preferred_element_type=jnp.float32).astype(jnp.bfloat16)
        sw_ref[pl.ds(k * _BK, _BK), :] = jnp.dot(
            s_ref[...].astype(jnp.bfloat16), wma_ref[...],
            preferred_element_type=jnp.float32).astype(jnp.bfloat16)

    part = jnp.dot(ar_ref[...].astype(jnp.bfloat16),
                   xw_ref[pl.ds(k * _BK, _BK), :],
                   preferred_element_type=jnp.float32)
    part = part + jnp.dot(ma_ref[...].astype(jnp.bfloat16),
                          sw_ref[pl.ds(k * _BK, _BK), :],
                          preferred_element_type=jnp.float32)

    @pl.when(k == 0)
    def _init():
        o_ref[...] = part + b_ref[...]

    @pl.when(k > 0)
    def _acc():
        o_ref[...] += part

    @pl.when(k == nk - 1)
    def _fin():
        o_ref[...] = jnp.maximum(o_ref[...], 0.0)


def kernel(x, arma_conv_AR, arma_conv_MA, input_signal, ar_kernel, ma_kernel, bias):
    n, f_in = x.shape
    f_out = ar_kernel.shape[1]
    nk = n // _BK

    war16 = ar_kernel.astype(jnp.bfloat16)
    wma16 = ma_kernel.astype(jnp.bfloat16)

    def _xs_map(i, k):
        return (jnp.where(i == 0, k, nk - 1), 0)

    out = pl.pallas_call(
        _body,
        grid=(n // _BM, nk),
        in_specs=[
            pl.BlockSpec((_BK, f_in), _xs_map),
            pl.BlockSpec((_BK, f_in), _xs_map),
            pl.BlockSpec((f_in, f_out), lambda i, k: (0, 0)),
            pl.BlockSpec((f_in, f_out), lambda i, k: (0, 0)),
            pl.BlockSpec((_BM, _BK), lambda i, k: (i, k)),
            pl.BlockSpec((_BM, _BK), lambda i, k: (i, k)),
            pl.BlockSpec((1, f_out), lambda i, k: (0, 0)),
        ],
        out_specs=pl.BlockSpec((_BM, f_out), lambda i, k: (i, 0)),
        out_shape=jax.ShapeDtypeStruct((n, f_out), jnp.float32),
        scratch_shapes=[
            pltpu.VMEM((n, f_out), jnp.bfloat16),
            pltpu.VMEM((n, f_out), jnp.bfloat16),
        ],
        compiler_params=pltpu.CompilerParams(
            dimension_semantics=("arbitrary", "arbitrary")),
    )(x, input_signal, war16, wma16, arma_conv_AR, arma_conv_MA,
      bias.reshape(1, f_out))

    return out


# speedup vs baseline: 1.3450x; 1.3450x over previous
import jax
import jax.numpy as jnp
from jax.experimental import pallas as pl
from jax.experimental.pallas import tpu as pltpu

_BK = 256


def _body(x_ref, s_ref, war_ref, wma_ref, ar_ref, ma_ref, b_ref, o_ref):
    k = pl.program_id(0)
    xw = jnp.dot(x_ref[...].astype(jnp.bfloat16), war_ref[...],
                 preferred_element_type=jnp.float32).astype(jnp.bfloat16)
    sw = jnp.dot(s_ref[...].astype(jnp.bfloat16), wma_ref[...],
                 preferred_element_type=jnp.float32).astype(jnp.bfloat16)
    part = jnp.dot(ar_ref[...].astype(jnp.bfloat16), xw,
                   preferred_element_type=jnp.float32)
    part = part + jnp.dot(ma_ref[...].astype(jnp.bfloat16), sw,
                          preferred_element_type=jnp.float32)

    @pl.when(k == 0)
    def _init():
        o_ref[...] = part + b_ref[...]

    @pl.when(k > 0)
    def _acc():
        o_ref[...] += part

    @pl.when(k == pl.num_programs(0) - 1)
    def _fin():
        o_ref[...] = jnp.maximum(o_ref[...], 0.0)


def kernel(x, arma_conv_AR, arma_conv_MA, input_signal, ar_kernel, ma_kernel, bias):
    n, f_in = x.shape
    f_out = ar_kernel.shape[1]

    war16 = ar_kernel.astype(jnp.bfloat16)
    wma16 = ma_kernel.astype(jnp.bfloat16)

    out = pl.pallas_call(
        _body,
        grid=(n // _BK,),
        in_specs=[
            pl.BlockSpec((_BK, f_in), lambda k: (k, 0)),
            pl.BlockSpec((_BK, f_in), lambda k: (k, 0)),
            pl.BlockSpec((f_in, f_out), lambda k: (0, 0)),
            pl.BlockSpec((f_in, f_out), lambda k: (0, 0)),
            pl.BlockSpec((n, _BK), lambda k: (0, k)),
            pl.BlockSpec((n, _BK), lambda k: (0, k)),
            pl.BlockSpec((1, f_out), lambda k: (0, 0)),
        ],
        out_specs=pl.BlockSpec((n, f_out), lambda k: (0, 0)),
        out_shape=jax.ShapeDtypeStruct((n, f_out), jnp.float32),
        compiler_params=pltpu.CompilerParams(
            dimension_semantics=("arbitrary",)),
    )(x, input_signal, war16, wma16, arma_conv_AR, arma_conv_MA,
      bias.reshape(1, f_out))

    return out


# speedup vs baseline: 1.5060x; 1.1197x over previous
import jax
import jax.numpy as jnp
from jax.experimental import pallas as pl
from jax.experimental.pallas import tpu as pltpu

_BM = 256


def _body(x_ref, s_ref, war_ref, wma_ref, ar_ref, ma_ref, b_ref, o_ref,
          xw_ref, sw_ref):
    i = pl.program_id(0)

    @pl.when(i == 0)
    def _premix():
        xw_ref[...] = jnp.dot(x_ref[...], war_ref[...],
                              preferred_element_type=jnp.float32,
                              precision=jax.lax.Precision.DEFAULT)
        sw_ref[...] = jnp.dot(s_ref[...], wma_ref[...],
                              preferred_element_type=jnp.float32,
                              precision=jax.lax.Precision.DEFAULT)

    acc = jnp.dot(ar_ref[...], xw_ref[...],
                  preferred_element_type=jnp.float32,
                  precision=jax.lax.Precision.DEFAULT)
    acc = acc + jnp.dot(ma_ref[...], sw_ref[...],
                        preferred_element_type=jnp.float32,
                        precision=jax.lax.Precision.DEFAULT)
    o_ref[...] = jnp.maximum(acc + b_ref[...], 0.0)


def kernel(x, arma_conv_AR, arma_conv_MA, input_signal, ar_kernel, ma_kernel, bias):
    n, f_in = x.shape
    f_out = ar_kernel.shape[1]

    out = pl.pallas_call(
        _body,
        grid=(n // _BM,),
        in_specs=[
            pl.BlockSpec((n, f_in), lambda i: (0, 0)),
            pl.BlockSpec((n, f_in), lambda i: (0, 0)),
            pl.BlockSpec((f_in, f_out), lambda i: (0, 0)),
            pl.BlockSpec((f_in, f_out), lambda i: (0, 0)),
            pl.BlockSpec((_BM, n), lambda i: (i, 0)),
            pl.BlockSpec((_BM, n), lambda i: (i, 0)),
            pl.BlockSpec((1, f_out), lambda i: (0, 0)),
        ],
        out_specs=pl.BlockSpec((_BM, f_out), lambda i: (i, 0)),
        out_shape=jax.ShapeDtypeStruct((n, f_out), jnp.float32),
        scratch_shapes=[
            pltpu.VMEM((n, f_out), jnp.float32),
            pltpu.VMEM((n, f_out), jnp.float32),
        ],
        compiler_params=pltpu.CompilerParams(
            dimension_semantics=("arbitrary",)),
    )(x, input_signal, ar_kernel, ma_kernel, arma_conv_AR, arma_conv_MA,
      bias.reshape(1, f_out))

    return out


# speedup vs baseline: 1.5235x; 1.0116x over previous
import jax
import jax.numpy as jnp
from jax.experimental import pallas as pl
from jax.experimental.pallas import tpu as pltpu

_BM = 256


def _body(x_ref, s_ref, war_ref, wma_ref, ar_ref, ma_ref, b_ref, o_ref,
          xw_ref, sw_ref):
    i = pl.program_id(0)

    @pl.when(i == 0)
    def _premix():
        xw_ref[...] = jnp.dot(x_ref[...], war_ref[...],
                              preferred_element_type=jnp.float32,
                              precision=jax.lax.Precision.DEFAULT
                              ).astype(jnp.bfloat16)
        sw_ref[...] = jnp.dot(s_ref[...], wma_ref[...],
                              preferred_element_type=jnp.float32,
                              precision=jax.lax.Precision.DEFAULT
                              ).astype(jnp.bfloat16)

    _dims = (((1,), (0,)), ((), ()))
    acc = jax.lax.dot_general(ar_ref[...], xw_ref[...], _dims,
                              preferred_element_type=jnp.float32,
                              precision=jax.lax.Precision.DEFAULT)
    acc = acc + jax.lax.dot_general(ma_ref[...], sw_ref[...], _dims,
                                    preferred_element_type=jnp.float32,
                                    precision=jax.lax.Precision.DEFAULT)
    o_ref[...] = jnp.maximum(acc + b_ref[...], 0.0)


def kernel(x, arma_conv_AR, arma_conv_MA, input_signal, ar_kernel, ma_kernel, bias):
    n, f_in = x.shape
    f_out = ar_kernel.shape[1]

    out = pl.pallas_call(
        _body,
        grid=(n // _BM,),
        in_specs=[
            pl.BlockSpec((n, f_in), lambda i: (0, 0)),
            pl.BlockSpec((n, f_in), lambda i: (0, 0)),
            pl.BlockSpec((f_in, f_out), lambda i: (0, 0)),
            pl.BlockSpec((f_in, f_out), lambda i: (0, 0)),
            pl.BlockSpec((_BM, n), lambda i: (i, 0)),
            pl.BlockSpec((_BM, n), lambda i: (i, 0)),
            pl.BlockSpec((1, f_out), lambda i: (0, 0)),
        ],
        out_specs=pl.BlockSpec((_BM, f_out), lambda i: (i, 0)),
        out_shape=jax.ShapeDtypeStruct((n, f_out), jnp.float32),
        scratch_shapes=[
            pltpu.VMEM((n, f_out), jnp.bfloat16),
            pltpu.VMEM((n, f_out), jnp.bfloat16),
        ],
        compiler_params=pltpu.CompilerParams(
            dimension_semantics=("arbitrary",)),
    )(x, input_signal, ar_kernel, ma_kernel, arma_conv_AR, arma_conv_MA,
      bias.reshape(1, f_out))

    return out
